# baseline (speedup 1.0000x reference)
import functools
import numpy as np
import jax
import jax.numpy as jnp
from jax import lax
from jax.experimental import pallas as pl
from jax.experimental.pallas import tpu as pltpu, tpu_sc as plsc

_VS = np.array([0.16, 0.16, 4.0], dtype=np.float32)
_LO = np.array([0.0, -39.68, -3.0], dtype=np.float32)
_HI = np.array([69.12, 39.68, 1.0], dtype=np.float32)
_GMIN = np.floor(_LO / _VS).astype(np.int32)
_GMAX = np.floor(_HI / _VS).astype(np.int32)
_DIMS = (_GMAX - _GMIN + 1).astype(np.int64)
BIG = int(_DIMS[0] * _DIMS[1] * _DIMS[2] + 1)
D0 = int(_DIMS[0])
D1 = int(_DIMS[1])
G0, G1, G2 = int(_GMIN[0]), int(_GMIN[1]), int(_GMIN[2])
MAXP = 32
MAXV = 16000
N = 150000

W = 32
CH = 4688
NP_ = W * CH
NV_A = CH // 16
OWNB = (BIG + 1 + W - 1) // W
CTRN = 13568
NCH_C = CTRN // 128
WINC = 2048
PAL_MAX = NP_ + W * 16
PDUMP_KB = PAL_MAX
PSZ = PAL_MAX + W * 64 + WINC
PILN = MAXV * MAXP
PIL_SZ = PILN + W * 64
VOUT = 16384
NPT_SZ = VOUT + 2048

mesh = plsc.VectorSubcoreMesh(core_axis_name="c", subcore_axis_name="s")
CP = pltpu.CompilerParams(needs_layout_passes=False,
                          use_tc_tiling_on_sc=False)
_f32 = jnp.float32
_i32 = jnp.int32


def _wid():
    return lax.axis_index("s") * 2 + lax.axis_index("c")


def _keys_of(x, y, z):
    in_r = (x >= 0.0) & (x <= 69.12) & (y >= -39.68) & (y <= 39.68) \
        & (z >= -3.0) & (z <= 1.0)

    def fdiv(v, d):
        q = v / _f32(d)
        t = q.astype(_i32)
        return t - (t.astype(_f32) > q).astype(_i32)

    cx = fdiv(x, 0.16)
    cy = fdiv(y, 0.16)
    cz = fdiv(z, 4.0)
    key = ((cz - G2) * D1 + (cy - G1)) * D0 + (cx - G0)
    return jnp.where(in_r, key, BIG)


def _owner_of(key):
    q = key.astype(_f32) / _f32(OWNB)
    return q.astype(_i32)


def _iota():
    return lax.iota(_i32, 16)


def _owner_bases(histb, w):
    zeros = jnp.zeros((16,), _i32)

    def body(wp, c):
        t0, t1, p0, p1 = c
        h0 = histb[pl.ds(wp * 32, 16)]
        h1 = histb[pl.ds(wp * 32 + 16, 16)]
        lt = (wp < w).astype(_i32)
        return (t0 + h0, t1 + h1, p0 + h0 * lt, p1 + h1 * lt)

    tot0, tot1, pf0, pf1 = lax.fori_loop(
        0, W, body, (zeros, zeros, zeros, zeros))
    at0 = (tot0 + 15) & -16
    at1 = (tot1 + 15) & -16
    e0 = plsc.cumsum(at0) - at0
    s0 = jnp.sum(at0)
    e1 = plsc.cumsum(at1) - at1 + s0
    return tot0, tot1, e0, e1, e0 + pf0, e1 + pf1


def _lane_of(v0, v1, w, scratch32):
    scratch32[pl.ds(0, 16)] = v0
    scratch32[pl.ds(16, 16)] = v1
    g = plsc.load_gather(scratch32, [jnp.full((16,), w, _i32)])
    return jnp.max(g)


@functools.partial(
    pl.kernel, mesh=mesh, compiler_params=CP, out_type=[],
    scratch_types=[pltpu.VMEM((CH, 4), _f32), pltpu.VMEM((CH,), _i32),
                   pltpu.VMEM((32,), _i32)])
def _ka(pc_hbm, keys_ref, hist_ref, ptb, keyb, h32):
    w = _wid()
    pltpu.sync_copy(pc_hbm.at[pl.ds(w * CH, CH)], ptb)
    h32[pl.ds(0, 16)] = jnp.zeros((16,), _i32)
    h32[pl.ds(16, 16)] = jnp.zeros((16,), _i32)

    def body(i, _):
        rows = i * 16 + _iota()
        x = plsc.load_gather(ptb, [rows, jnp.full((16,), 0, _i32)])
        y = plsc.load_gather(ptb, [rows, jnp.full((16,), 1, _i32)])
        z = plsc.load_gather(ptb, [rows, jnp.full((16,), 2, _i32)])
        key = _keys_of(x, y, z)
        keyb[pl.ds(i * 16, 16)] = key
        own = _owner_of(key)
        cnt, last = plsc.scan_count(own)
        prev = plsc.load_gather(h32, [own], mask=last)
        plsc.store_scatter(h32, [own], prev + cnt, mask=last)
        return 0

    lax.fori_loop(0, NV_A, body, 0)
    pltpu.sync_copy(keyb, keys_ref.at[pl.ds(w * CH, CH)])
    pltpu.sync_copy(h32, hist_ref.at[pl.ds(w * 32, 32)])


@functools.partial(
    pl.kernel, mesh=mesh, compiler_params=CP, out_type=[],
    scratch_types=[pltpu.VMEM((W * W,), _i32), pltpu.VMEM((CH,), _i32),
                   pltpu.VMEM((CH + 48,), _i32), pltpu.VMEM((37, 128), _i32),
                   pltpu.VMEM((32,), _i32), pltpu.SemaphoreType.DMA])
def _kb(keys_ref, hist_ref, pkey_ref, histb, keyb, valb, destb, offb, sem):
    w = _wid()
    pltpu.sync_copy(hist_ref, histb)
    pltpu.sync_copy(keys_ref.at[pl.ds(w * CH, CH)], keyb)
    _, _, _, _, db0, db1 = _owner_bases(histb, w)
    offb[pl.ds(0, 16)] = db0
    offb[pl.ds(16, 16)] = db1

    def body(i, _):
        k = keyb[pl.ds(i * 16, 16)]
        own = _owner_of(k)
        cnt, last = plsc.scan_count(own)
        prev = plsc.load_gather(offb, [own])
        dest = prev + cnt - 1
        plsc.store_scatter(offb, [own], prev + cnt, mask=last)
        r = i >> 3
        cc = (i & 7) * 16
        destb[r, pl.ds(cc, 16)] = dest
        loc = k - own * OWNB
        valb[pl.ds(i * 16, 16)] = (loc << 18) | (w * CH + i * 16 + _iota())
        return 0

    lax.fori_loop(0, NV_A, body, 0)
    for j in range(3):
        valb[pl.ds(CH + j * 16, 16)] = jnp.zeros((16,), _i32)
        destb[36, pl.ds(80 + j * 16, 16)] = \
            PDUMP_KB + w * 64 + j * 16 + _iota()
    descs = []
    for c in range(37):
        descs.append(pltpu.async_copy(
            valb.at[pl.ds(c * 128, 128)], pkey_ref.at[destb.at[c]], sem))
        if len(descs) == 16:
            for d in descs:
                d.wait()
            descs = []
    for d in descs:
        d.wait()


@functools.partial(
    pl.kernel, mesh=mesh, compiler_params=CP, out_type=[],
    scratch_types=[pltpu.VMEM((W * W,), _i32), pltpu.VMEM((CTRN,), _i32),
                   pltpu.VMEM((WINC,), _i32), pltpu.VMEM((32,), _i32),
                   pltpu.VMEM((16,), _i32)])
def _kc(hist_ref, pkey_ref, counts_ref, occ_ref, histb, ctrb, kwin, s32, s16):
    w = _wid()
    pltpu.sync_copy(hist_ref, histb)
    tot0, tot1, e0, e1, _, _ = _owner_bases(histb, w)
    n_w = _lane_of(tot0, tot1, w, s32)
    pb_w = pl.multiple_of(_lane_of(e0, e1, w, s32), 16)

    def zbody(i, _):
        ctrb[pl.ds(i * 16, 16)] = jnp.zeros((16,), _i32)
        return 0

    lax.fori_loop(0, CTRN // 16, zbody, 0)
    nwin = (n_w + (WINC - 1)) >> 11

    def wloop(t, _):
        pltpu.sync_copy(pkey_ref.at[pl.ds(pb_w + t * WINC, WINC)], kwin)

        def vloop(i, _):
            j0 = t * WINC + i * 16
            mask = (j0 + _iota()) < n_w
            p = kwin[pl.ds(i * 16, 16)]
            loc = jnp.minimum(lax.shift_right_logical(p, 18), CTRN - 1)
            cnt, last = plsc.scan_count(loc, mask)
            ml = mask & last
            prev = plsc.load_gather(ctrb, [loc], mask=ml)
            plsc.store_scatter(ctrb, [loc], prev + cnt, mask=ml)
            return 0

        lax.fori_loop(0, WINC // 16, vloop, 0)
        return 0

    lax.fori_loop(0, nwin, wloop, 0)
    pltpu.sync_copy(ctrb, counts_ref.at[pl.ds(w * CTRN, CTRN)])

    def oloop(i, acc):
        cnt = ctrb[pl.ds(i * 16, 16)]
        gb = w * OWNB + i * 16 + _iota()
        om = (cnt > 0) & (gb < BIG)
        return acc + plsc.all_reduce_population_count(om)

    occv = lax.fori_loop(0, CTRN // 16, oloop, jnp.zeros((16,), _i32))
    s16[...] = occv
    pltpu.sync_copy(s16, occ_ref.at[pl.ds(w * 16, 16)])


@functools.partial(
    pl.kernel, mesh=mesh, compiler_params=CP, out_type=[],
    scratch_types=[pltpu.VMEM((W * W,), _i32), pltpu.VMEM((W * 16,), _i32),
                   pltpu.VMEM((CTRN,), _i32), pltpu.VMEM((CTRN,), _i32),
                   pltpu.VMEM((NCH_C, 128), _i32), pltpu.VMEM((CTRN,), _i32),
                   pltpu.VMEM((CTRN,), _i32), pltpu.VMEM((WINC,), _i32),
                   pltpu.VMEM((16, 128), _i32),
                   pltpu.VMEM((16, 128), _i32), pltpu.VMEM((WINC, 16), _f32),
                   pltpu.VMEM((32,), _i32), pltpu.SemaphoreType.DMA])
def _ke(pc_hbm, hist_ref, pkey_ref, counts_ref, occ_ref,
        pil_ref, npt_ref, km_ref,
        histb, occb, ctrb, vmb, didxb, npvb, kmvb, kwin, gib,
        destb, pcb, s32, sem):
    w = _wid()
    pltpu.sync_copy(hist_ref, histb)
    pltpu.sync_copy(occ_ref, occb)
    tot0, tot1, e0, e1, _, _ = _owner_bases(histb, w)
    n_w = _lane_of(tot0, tot1, w, s32)
    pb_w = pl.multiple_of(_lane_of(e0, e1, w, s32), 16)
    pltpu.sync_copy(counts_ref.at[pl.ds(w * CTRN, CTRN)], ctrb)
    o0 = plsc.load_gather(occb, [_iota() * 16])
    o1 = plsc.load_gather(occb, [(_iota() + 16) * 16])
    io = _iota()
    base_w = jnp.sum(jnp.where(io < w, o0, 0)) + \
        jnp.sum(jnp.where(io + 16 < w, o1, 0))

    def vml(i, carry):
        vc, maxc = carry
        cnt = ctrb[pl.ds(i * 16, 16)]
        gb = w * OWNB + i * 16 + _iota()
        om = (cnt > 0) & (gb < BIG)
        omi = om.astype(_i32)
        excl = plsc.cumsum(omi) - omi
        vid = vc + excl
        vmb[pl.ds(i * 16, 16)] = vid
        sel = om & (vid < MAXV)
        didx = jnp.where(sel, vid, VOUT + (i & 127) * 16 + _iota())
        r = i >> 3
        cc = (i & 7) * 16
        didxb[r, pl.ds(cc, 16)] = didx
        npvb[pl.ds(i * 16, 16)] = jnp.minimum(cnt, MAXP)
        kmvb[pl.ds(i * 16, 16)] = gb
        nsel = plsc.all_reduce_population_count(sel)
        maxc = jnp.where(nsel > 0, jnp.maximum(maxc, i >> 3), maxc)
        return vc + plsc.all_reduce_population_count(om), maxc

    _, maxc = lax.fori_loop(
        0, CTRN // 16, vml,
        (jnp.full((16,), base_w, _i32), jnp.full((16,), -1, _i32)))

    def fire(b, _):
        for u in range(8):
            c = jnp.minimum(b * 8 + u, NCH_C - 1)
            pltpu.async_copy(
                npvb.at[pl.ds(c * 128, 128)], npt_ref.at[didxb.at[c]], sem)
            pltpu.async_copy(
                kmvb.at[pl.ds(c * 128, 128)], km_ref.at[didxb.at[c]], sem)
        for u in range(8):
            c = jnp.minimum(b * 8 + u, NCH_C - 1)
            pltpu.make_async_copy(
                npvb.at[pl.ds(c * 128, 128)], npt_ref.at[didxb.at[c]],
                sem).wait()
            pltpu.make_async_copy(
                kmvb.at[pl.ds(c * 128, 128)], km_ref.at[didxb.at[c]],
                sem).wait()
        return 0

    lax.fori_loop(0, (jnp.max(maxc) + 8) >> 3, fire, 0)

    def zbody(i, _):
        ctrb[pl.ds(i * 16, 16)] = jnp.zeros((16,), _i32)
        return 0

    lax.fori_loop(0, CTRN // 16, zbody, 0)

    def pf(i, _):
        r = i >> 3
        cc = (i & 7) * 16
        destb[r, pl.ds(cc, 16)] = PILN + w * 64 + (i & 3) * 16 + _iota()
        gib[r, pl.ds(cc, 16)] = jnp.zeros((16,), _i32)
        return 0

    lax.fori_loop(0, WINC // 16, pf, 0)
    nwin = (n_w + (WINC - 1)) >> 11

    bigloc = BIG - w * OWNB

    def wloop(t, _):
        pltpu.sync_copy(pkey_ref.at[pl.ds(pb_w + t * WINC, WINC)], kwin)

        def vloop(i, oc):
            j0 = t * WINC + i * 16
            mask = (j0 + _iota()) < n_w
            p = kwin[pl.ds(i * 16, 16)]
            loc = jnp.minimum(lax.shift_right_logical(p, 18), CTRN - 1)
            cnt, last = plsc.scan_count(loc, mask)
            prev = plsc.load_gather(ctrb, [loc], mask=mask)
            rank = prev + cnt - 1
            plsc.store_scatter(ctrb, [loc], prev + cnt, mask=mask & last)
            vid = plsc.load_gather(vmb, [loc], mask=mask)
            valid = mask & (loc != bigloc) & (rank < MAXP) & (vid < MAXV)
            vi = valid.astype(_i32)
            rk = oc + plsc.cumsum(vi) - vi
            plsc.store_scatter(destb, [rk >> 7, rk & 127],
                               vid * MAXP + rank, mask=valid)
            giv = jnp.minimum(p & 0x3FFFF, NP_ - 1)
            plsc.store_scatter(gib, [rk >> 7, rk & 127], giv, mask=valid)
            return oc + plsc.all_reduce_population_count(valid)

        oc = lax.fori_loop(0, WINC // 16, vloop, jnp.zeros((16,), _i32))
        ntr = (jnp.max(oc) + 127) >> 7

        def dloop(c, _):
            pltpu.async_copy(pc_hbm.at[gib.at[c]],
                             pcb.at[pl.ds(c * 128, 128)], sem).wait()
            pltpu.async_copy(pcb.at[pl.ds(c * 128, 128)],
                             pil_ref.at[destb.at[c]], sem).wait()
            return 0

        lax.fori_loop(0, ntr, dloop, 0)
        return 0

    lax.fori_loop(0, nwin, wloop, 0)


@functools.partial(
    pl.kernel, mesh=mesh, compiler_params=CP,
    out_type=jax.ShapeDtypeStruct((VOUT, 4), _i32),
    scratch_types=[pltpu.VMEM((512,), _i32), pltpu.VMEM((512, 4), _i32)])
def _kf(km_ref, out_ref, kb, cb):
    w = _wid()
    pltpu.sync_copy(km_ref.at[pl.ds(w * 512, 512)], kb)

    def body(i, _):
        rows = i * 16 + _iota()
        k = kb[pl.ds(i * 16, 16)]
        ok = k >= 0
        kc = jnp.maximum(k, 0)
        q = (kc.astype(_f32) / _f32(D0)).astype(_i32)
        cx = kc - q * D0
        q2 = (q.astype(_f32) / _f32(D1)).astype(_i32)
        cy = q - q2 * D1
        oki = ok.astype(_i32)
        plsc.store_scatter(cb, [rows, jnp.full((16,), 0, _i32)],
                           (cx + G0) * oki)
        plsc.store_scatter(cb, [rows, jnp.full((16,), 1, _i32)],
                           (cy + G1) * oki)
        plsc.store_scatter(cb, [rows, jnp.full((16,), 2, _i32)],
                           (q2 + G2) * oki)
        plsc.store_scatter(cb, [rows, jnp.full((16,), 3, _i32)], oki * 0)
        return 0

    lax.fori_loop(0, 32, body, 0)
    pltpu.sync_copy(cb, out_ref.at[pl.ds(w * 512, 512)])


def kernel(pointcloud):
    pc = jnp.concatenate(
        [pointcloud, jnp.full((NP_ - N, 4), 1e4, _f32)], axis=0)
    keys_ref = jax.new_ref(jnp.zeros((NP_,), _i32))
    hist_ref = jax.new_ref(jnp.zeros((W * W,), _i32))
    pkey_ref = jax.new_ref(jnp.zeros((PSZ,), _i32))
    counts_ref = jax.new_ref(jnp.zeros((W * CTRN,), _i32))
    occ_ref = jax.new_ref(jnp.zeros((W * 16,), _i32))
    pil_ref = jax.new_ref(jnp.zeros((PIL_SZ, 16), _f32))
    npt_ref = jax.new_ref(jnp.zeros((NPT_SZ,), _i32))
    km_ref = jax.new_ref(jnp.full((NPT_SZ,), -1, _i32))
    pc16 = jnp.pad(pc, ((0, 0), (0, 12)))
    _ka(pc, keys_ref, hist_ref)
    _kb(keys_ref, hist_ref, pkey_ref)
    _kc(hist_ref, pkey_ref, counts_ref, occ_ref)
    _ke(pc16, hist_ref, pkey_ref, counts_ref, occ_ref,
        pil_ref, npt_ref, km_ref)
    coords4 = _kf(km_ref)
    pillars = pil_ref[...][:PILN].reshape(MAXV, MAXP, 16)[..., :4]
    pillar_coords = coords4[:MAXV, :3]
    num_points = npt_ref[...][:MAXV]
    return pillars, pillar_coords, num_points


# speedup vs baseline: 1.3965x; 1.3965x over previous
import functools
import numpy as np
import jax
import jax.numpy as jnp
from jax import lax
from jax.experimental import pallas as pl
from jax.experimental.pallas import tpu as pltpu, tpu_sc as plsc

_VS = np.array([0.16, 0.16, 4.0], dtype=np.float32)
_LO = np.array([0.0, -39.68, -3.0], dtype=np.float32)
_HI = np.array([69.12, 39.68, 1.0], dtype=np.float32)
_GMIN = np.floor(_LO / _VS).astype(np.int32)
_GMAX = np.floor(_HI / _VS).astype(np.int32)
_DIMS = (_GMAX - _GMIN + 1).astype(np.int64)
BIG = int(_DIMS[0] * _DIMS[1] * _DIMS[2] + 1)
D0 = int(_DIMS[0])
D1 = int(_DIMS[1])
G0, G1, G2 = int(_GMIN[0]), int(_GMIN[1]), int(_GMIN[2])
MAXP = 32
MAXV = 16000
N = 150000

W = 32
CH = 4688
NP_ = W * CH
NV_A = CH // 16
OWNB = (BIG + 1 + W - 1) // W
CTRN = 13568
NCH_C = CTRN // 128
WINC = 2048
PAL_MAX = NP_ + W * 16
PDUMP_KB = PAL_MAX
PSZ = PAL_MAX + W * 64 + WINC
PILN = MAXV * MAXP
PIL_SZ = PILN + W * 64
VOUT = 16384
NPT_SZ = VOUT + 2048

mesh = plsc.VectorSubcoreMesh(core_axis_name="c", subcore_axis_name="s")
CP = pltpu.CompilerParams(needs_layout_passes=False,
                          use_tc_tiling_on_sc=False)
_f32 = jnp.float32
_i32 = jnp.int32


def _wid():
    return lax.axis_index("s") * 2 + lax.axis_index("c")


def _keys_of(x, y, z):
    in_r = (x >= 0.0) & (x <= 69.12) & (y >= -39.68) & (y <= 39.68) \
        & (z >= -3.0) & (z <= 1.0)

    def fdiv(v, d):
        q = v / _f32(d)
        t = q.astype(_i32)
        return t - (t.astype(_f32) > q).astype(_i32)

    cx = fdiv(x, 0.16)
    cy = fdiv(y, 0.16)
    cz = fdiv(z, 4.0)
    key = ((cz - G2) * D1 + (cy - G1)) * D0 + (cx - G0)
    return jnp.where(in_r, key, BIG)


def _owner_of(key):
    q = key.astype(_f32) / _f32(OWNB)
    return q.astype(_i32)


def _iota():
    return lax.iota(_i32, 16)


def _owner_bases(histb, w):
    zeros = jnp.zeros((16,), _i32)

    def body(wp, c):
        t0, t1, p0, p1 = c
        h0 = histb[pl.ds(wp * 32, 16)]
        h1 = histb[pl.ds(wp * 32 + 16, 16)]
        lt = (wp < w).astype(_i32)
        return (t0 + h0, t1 + h1, p0 + h0 * lt, p1 + h1 * lt)

    tot0, tot1, pf0, pf1 = lax.fori_loop(
        0, W, body, (zeros, zeros, zeros, zeros))
    at0 = (tot0 + 15) & -16
    at1 = (tot1 + 15) & -16
    e0 = plsc.cumsum(at0) - at0
    s0 = jnp.sum(at0)
    e1 = plsc.cumsum(at1) - at1 + s0
    return tot0, tot1, e0, e1, e0 + pf0, e1 + pf1


def _lane_of(v0, v1, w, scratch32):
    scratch32[pl.ds(0, 16)] = v0
    scratch32[pl.ds(16, 16)] = v1
    g = plsc.load_gather(scratch32, [jnp.full((16,), w, _i32)])
    return jnp.max(g)


@functools.partial(
    pl.kernel, mesh=mesh, compiler_params=CP, out_type=[],
    scratch_types=[pltpu.VMEM((CH, 4), _f32), pltpu.VMEM((CH,), _i32),
                   pltpu.VMEM((32,), _i32)])
def _ka(pc_hbm, keys_ref, hist_ref, ptb, keyb, h32):
    w = _wid()
    pltpu.sync_copy(pc_hbm.at[pl.ds(w * CH, CH)], ptb)
    h32[pl.ds(0, 16)] = jnp.zeros((16,), _i32)
    h32[pl.ds(16, 16)] = jnp.zeros((16,), _i32)

    def body(i, _):
        rows = i * 16 + _iota()
        x = plsc.load_gather(ptb, [rows, jnp.full((16,), 0, _i32)])
        y = plsc.load_gather(ptb, [rows, jnp.full((16,), 1, _i32)])
        z = plsc.load_gather(ptb, [rows, jnp.full((16,), 2, _i32)])
        key = _keys_of(x, y, z)
        keyb[pl.ds(i * 16, 16)] = key
        own = _owner_of(key)
        cnt, last = plsc.scan_count(own)
        prev = plsc.load_gather(h32, [own], mask=last)
        plsc.store_scatter(h32, [own], prev + cnt, mask=last)
        return 0

    lax.fori_loop(0, NV_A, body, 0)
    pltpu.sync_copy(keyb, keys_ref.at[pl.ds(w * CH, CH)])
    pltpu.sync_copy(h32, hist_ref.at[pl.ds(w * 32, 32)])


@functools.partial(
    pl.kernel, mesh=mesh, compiler_params=CP, out_type=[],
    scratch_types=[pltpu.VMEM((W * W,), _i32), pltpu.VMEM((CH,), _i32),
                   pltpu.VMEM((CH + 48,), _i32), pltpu.VMEM((37, 128), _i32),
                   pltpu.VMEM((32,), _i32), pltpu.SemaphoreType.DMA])
def _kb(keys_ref, hist_ref, pkey_ref, histb, keyb, valb, destb, offb, sem):
    w = _wid()
    pltpu.sync_copy(hist_ref, histb)
    pltpu.sync_copy(keys_ref.at[pl.ds(w * CH, CH)], keyb)
    _, _, _, _, db0, db1 = _owner_bases(histb, w)
    offb[pl.ds(0, 16)] = db0
    offb[pl.ds(16, 16)] = db1

    def body(i, _):
        k = keyb[pl.ds(i * 16, 16)]
        own = _owner_of(k)
        cnt, last = plsc.scan_count(own)
        prev = plsc.load_gather(offb, [own])
        dest = prev + cnt - 1
        plsc.store_scatter(offb, [own], prev + cnt, mask=last)
        r = i >> 3
        cc = (i & 7) * 16
        destb[r, pl.ds(cc, 16)] = dest
        loc = k - own * OWNB
        valb[pl.ds(i * 16, 16)] = (loc << 18) | (w * CH + i * 16 + _iota())
        return 0

    lax.fori_loop(0, NV_A, body, 0)
    for j in range(3):
        valb[pl.ds(CH + j * 16, 16)] = jnp.zeros((16,), _i32)
        destb[36, pl.ds(80 + j * 16, 16)] = \
            PDUMP_KB + w * 64 + j * 16 + _iota()
    descs = []
    for c in range(37):
        descs.append(pltpu.async_copy(
            valb.at[pl.ds(c * 128, 128)], pkey_ref.at[destb.at[c]], sem))
        if len(descs) == 16:
            for d in descs:
                d.wait()
            descs = []
    for d in descs:
        d.wait()


@functools.partial(
    pl.kernel, mesh=mesh, compiler_params=CP, out_type=[],
    scratch_types=[pltpu.VMEM((W * W,), _i32), pltpu.VMEM((CTRN,), _i32),
                   pltpu.VMEM((WINC,), _i32), pltpu.VMEM((32,), _i32),
                   pltpu.VMEM((16,), _i32)])
def _kc(hist_ref, pkey_ref, counts_ref, occ_ref, histb, ctrb, kwin, s32, s16):
    w = _wid()
    pltpu.sync_copy(hist_ref, histb)
    tot0, tot1, e0, e1, _, _ = _owner_bases(histb, w)
    n_w = _lane_of(tot0, tot1, w, s32)
    pb_w = pl.multiple_of(_lane_of(e0, e1, w, s32), 16)

    def zbody(i, _):
        ctrb[pl.ds(i * 16, 16)] = jnp.zeros((16,), _i32)
        return 0

    lax.fori_loop(0, CTRN // 16, zbody, 0)
    nwin = (n_w + (WINC - 1)) >> 11

    def wloop(t, _):
        pltpu.sync_copy(pkey_ref.at[pl.ds(pb_w + t * WINC, WINC)], kwin)

        def vloop(i, _):
            j0 = t * WINC + i * 16
            mask = (j0 + _iota()) < n_w
            p = kwin[pl.ds(i * 16, 16)]
            loc = jnp.minimum(lax.shift_right_logical(p, 18), CTRN - 1)
            cnt, last = plsc.scan_count(loc, mask)
            ml = mask & last
            prev = plsc.load_gather(ctrb, [loc], mask=ml)
            plsc.store_scatter(ctrb, [loc], prev + cnt, mask=ml)
            return 0

        lax.fori_loop(0, WINC // 16, vloop, 0)
        return 0

    lax.fori_loop(0, nwin, wloop, 0)
    pltpu.sync_copy(ctrb, counts_ref.at[pl.ds(w * CTRN, CTRN)])

    def oloop(i, acc):
        cnt = ctrb[pl.ds(i * 16, 16)]
        gb = w * OWNB + i * 16 + _iota()
        om = (cnt > 0) & (gb < BIG)
        return acc + plsc.all_reduce_population_count(om)

    occv = lax.fori_loop(0, CTRN // 16, oloop, jnp.zeros((16,), _i32))
    s16[...] = occv
    pltpu.sync_copy(s16, occ_ref.at[pl.ds(w * 16, 16)])


@functools.partial(
    pl.kernel, mesh=mesh, compiler_params=CP, out_type=[],
    scratch_types=[pltpu.VMEM((W * W,), _i32), pltpu.VMEM((W * 16,), _i32),
                   pltpu.VMEM((CTRN,), _i32), pltpu.VMEM((CTRN,), _i32),
                   pltpu.VMEM((NCH_C + 1, 128), _i32),
                   pltpu.VMEM((CTRN + 128,), _i32),
                   pltpu.VMEM((CTRN + 128,), _i32), pltpu.VMEM((WINC,), _i32),
                   pltpu.VMEM((16, 128), _i32),
                   pltpu.VMEM((16, 128), _i32), pltpu.VMEM((WINC, 16), _f32),
                   pltpu.VMEM((32,), _i32), pltpu.SemaphoreType.DMA,
                   pltpu.SemaphoreType.DMA])
def _ke(pc_hbm, hist_ref, pkey_ref, counts_ref, occ_ref,
        pil_ref, npt_ref, km_ref,
        histb, occb, ctrb, vmb, didxb, npvb, kmvb, kwin, gib,
        destb, pcb, s32, sem, sem2):
    w = _wid()
    pltpu.sync_copy(hist_ref, histb)
    pltpu.sync_copy(occ_ref, occb)
    tot0, tot1, e0, e1, _, _ = _owner_bases(histb, w)
    n_w = _lane_of(tot0, tot1, w, s32)
    pb_w = pl.multiple_of(_lane_of(e0, e1, w, s32), 16)
    pltpu.sync_copy(counts_ref.at[pl.ds(w * CTRN, CTRN)], ctrb)
    o0 = plsc.load_gather(occb, [_iota() * 16])
    o1 = plsc.load_gather(occb, [(_iota() + 16) * 16])
    io = _iota()
    base_w = jnp.sum(jnp.where(io < w, o0, 0)) + \
        jnp.sum(jnp.where(io + 16 < w, o1, 0))

    @pl.when(base_w < MAXV)
    def _work():
        def vml(i, carry):
            vc, ocn = carry
            cnt = ctrb[pl.ds(i * 16, 16)]
            gb = w * OWNB + i * 16 + _iota()
            om = (cnt > 0) & (gb < BIG)
            omi = om.astype(_i32)
            excl = plsc.cumsum(omi) - omi
            vid = vc + excl
            vmb[pl.ds(i * 16, 16)] = vid
            sel = om & (vid < MAXV)
            si = sel.astype(_i32)
            rk2 = ocn + plsc.cumsum(si) - si
            plsc.store_scatter(didxb, [rk2 >> 7, rk2 & 127], vid, mask=sel)
            plsc.store_scatter(npvb, [rk2], jnp.minimum(cnt, MAXP), mask=sel)
            plsc.store_scatter(kmvb, [rk2], gb, mask=sel)
            return (vc + plsc.all_reduce_population_count(om),
                    ocn + plsc.all_reduce_population_count(sel))

        _, ocn = lax.fori_loop(
            0, CTRN // 16, vml,
            (jnp.full((16,), base_w, _i32), jnp.zeros((16,), _i32)))
        for u in range(8):
            rkp = ocn + u * 16 + _iota()
            plsc.store_scatter(didxb, [rkp >> 7, rkp & 127],
                               VOUT + (w & 15) * 128 + u * 16 + _iota())

        def fire(c, _):
            pltpu.async_copy(
                npvb.at[pl.ds(c * 128, 128)], npt_ref.at[didxb.at[c]], sem)
            pltpu.async_copy(
                kmvb.at[pl.ds(c * 128, 128)], km_ref.at[didxb.at[c]], sem)
            pltpu.make_async_copy(
                npvb.at[pl.ds(c * 128, 128)], npt_ref.at[didxb.at[c]],
                sem).wait()
            pltpu.make_async_copy(
                kmvb.at[pl.ds(c * 128, 128)], km_ref.at[didxb.at[c]],
                sem).wait()
            return 0

        lax.fori_loop(0, (jnp.max(ocn) + 127) >> 7, fire, 0)

        def zbody(i, _):
            ctrb[pl.ds(i * 16, 16)] = jnp.zeros((16,), _i32)
            return 0

        lax.fori_loop(0, CTRN // 16, zbody, 0)

        def pf(i, _):
            r = i >> 3
            cc = (i & 7) * 16
            destb[r, pl.ds(cc, 16)] = PILN + w * 64 + (i & 3) * 16 + _iota()
            gib[r, pl.ds(cc, 16)] = jnp.zeros((16,), _i32)
            return 0

        lax.fori_loop(0, WINC // 16, pf, 0)
        nwin = (n_w + (WINC - 1)) >> 11
        bigloc = BIG - w * OWNB

        def wloop(t, _):
            pltpu.sync_copy(pkey_ref.at[pl.ds(pb_w + t * WINC, WINC)], kwin)

            def vloop(i, oc):
                j0 = t * WINC + i * 16
                mask = (j0 + _iota()) < n_w
                p = kwin[pl.ds(i * 16, 16)]
                loc = jnp.minimum(lax.shift_right_logical(p, 18), CTRN - 1)
                cnt, last = plsc.scan_count(loc, mask)
                prev = plsc.load_gather(ctrb, [loc], mask=mask)
                rank = prev + cnt - 1
                plsc.store_scatter(ctrb, [loc], prev + cnt, mask=mask & last)
                vid = plsc.load_gather(vmb, [loc], mask=mask)
                valid = mask & (loc != bigloc) & (rank < MAXP) & (vid < MAXV)
                vi = valid.astype(_i32)
                rk = oc + plsc.cumsum(vi) - vi
                plsc.store_scatter(destb, [rk >> 7, rk & 127],
                                   vid * MAXP + rank, mask=valid)
                giv = jnp.minimum(p & 0x3FFFF, NP_ - 1)
                plsc.store_scatter(gib, [rk >> 7, rk & 127], giv, mask=valid)
                return oc + plsc.all_reduce_population_count(valid)

            oc = lax.fori_loop(0, WINC // 16, vloop, jnp.zeros((16,), _i32))
            ntr = (jnp.max(oc) + 127) >> 7

            def dloop(c, _):
                pltpu.async_copy(pc_hbm.at[gib.at[c]],
                                 pcb.at[pl.ds(c * 128, 128)], sem).wait()
                pltpu.async_copy(pcb.at[pl.ds(c * 128, 128)],
                                 pil_ref.at[destb.at[c]], sem2)
                return 0

            lax.fori_loop(0, ntr, dloop, 0)

            def drain(c, _):
                pltpu.make_async_copy(pcb.at[pl.ds(c * 128, 128)],
                                      pil_ref.at[destb.at[c]], sem2).wait()
                return 0

            lax.fori_loop(0, ntr, drain, 0)
            return 0

        lax.fori_loop(0, nwin, wloop, 0)


@functools.partial(
    pl.kernel, mesh=mesh, compiler_params=CP,
    out_type=jax.ShapeDtypeStruct((VOUT, 4), _i32),
    scratch_types=[pltpu.VMEM((512,), _i32), pltpu.VMEM((512, 4), _i32)])
def _kf(km_ref, out_ref, kb, cb):
    w = _wid()
    pltpu.sync_copy(km_ref.at[pl.ds(w * 512, 512)], kb)

    def body(i, _):
        rows = i * 16 + _iota()
        k = kb[pl.ds(i * 16, 16)]
        ok = k >= 0
        kc = jnp.maximum(k, 0)
        q = (kc.astype(_f32) / _f32(D0)).astype(_i32)
        cx = kc - q * D0
        q2 = (q.astype(_f32) / _f32(D1)).astype(_i32)
        cy = q - q2 * D1
        oki = ok.astype(_i32)
        plsc.store_scatter(cb, [rows, jnp.full((16,), 0, _i32)],
                           (cx + G0) * oki)
        plsc.store_scatter(cb, [rows, jnp.full((16,), 1, _i32)],
                           (cy + G1) * oki)
        plsc.store_scatter(cb, [rows, jnp.full((16,), 2, _i32)],
                           (q2 + G2) * oki)
        plsc.store_scatter(cb, [rows, jnp.full((16,), 3, _i32)], oki * 0)
        return 0

    lax.fori_loop(0, 32, body, 0)
    pltpu.sync_copy(cb, out_ref.at[pl.ds(w * 512, 512)])


def kernel(pointcloud):
    pc = jnp.concatenate(
        [pointcloud, jnp.full((NP_ - N, 4), 1e4, _f32)], axis=0)
    keys_ref = jax.new_ref(jnp.zeros((NP_,), _i32))
    hist_ref = jax.new_ref(jnp.zeros((W * W,), _i32))
    pkey_ref = jax.new_ref(jnp.zeros((PSZ,), _i32))
    counts_ref = jax.new_ref(jnp.zeros((W * CTRN,), _i32))
    occ_ref = jax.new_ref(jnp.zeros((W * 16,), _i32))
    pil_ref = jax.new_ref(jnp.zeros((PIL_SZ, 16), _f32))
    npt_ref = jax.new_ref(jnp.zeros((NPT_SZ,), _i32))
    km_ref = jax.new_ref(jnp.full((NPT_SZ,), -1, _i32))
    pc16 = jnp.pad(pc, ((0, 0), (0, 12)))
    _ka(pc, keys_ref, hist_ref)
    _kb(keys_ref, hist_ref, pkey_ref)
    _kc(hist_ref, pkey_ref, counts_ref, occ_ref)
    _ke(pc16, hist_ref, pkey_ref, counts_ref, occ_ref,
        pil_ref, npt_ref, km_ref)
    coords4 = _kf(km_ref)
    pillars = pil_ref[...][:PILN].reshape(MAXV, MAXP, 16)[..., :4]
    pillar_coords = coords4[:MAXV, :3]
    num_points = npt_ref[...][:MAXV]
    return pillars, pillar_coords, num_points
